# initial kernel scaffold (unmeasured)
import jax
import jax.numpy as jnp
from jax import lax
from jax.experimental import pallas as pl
from jax.experimental.pallas import tpu as pltpu


def kernel(
    t,
):
    def body(*refs):
        pass

    out_shape = jax.ShapeDtypeStruct(..., jnp.float32)
    return pl.pallas_call(body, out_shape=out_shape)(...)


# baseline (device time: 602934 ns/iter reference)
import jax
import jax.numpy as jnp
from jax import lax
from jax.experimental import pallas as pl
from jax.experimental.pallas import tpu as pltpu

N_DEV = 4
CHUNK = 2048
N_COLS = 2048


def kernel(t):
    m_per, n = t.shape
    assert m_per == N_DEV * CHUNK and n == N_COLS

    def body(x_hbm, out_hbm, comm, xstage, stage_sem, out_sem,
             rs_send, rs_recv, ag_send, ag_recv, credit):
        my = lax.axis_index("i")
        left = (my - 1) % N_DEV
        right = (my + 1) % N_DEV

        barrier_sem = pltpu.get_barrier_semaphore()
        for nbr in (left, right):
            pl.semaphore_signal(
                barrier_sem, inc=1,
                device_id=(nbr,), device_id_type=pl.DeviceIdType.MESH,
            )
        pl.semaphore_wait(barrier_sem, 2)

        cp = pltpu.make_async_copy(
            x_hbm.at[pl.ds(my * CHUNK, CHUNK), :], xstage, stage_sem)
        cp.start()
        cp.wait()
        comm[0, :, :] = xstage[:, :].astype(jnp.bfloat16)

        for s in range(N_DEV - 1):
            rdma = pltpu.make_async_remote_copy(
                src_ref=comm.at[s],
                dst_ref=comm.at[s + 1],
                send_sem=rs_send.at[s],
                recv_sem=rs_recv.at[s],
                device_id=(right,),
                device_id_type=pl.DeviceIdType.MESH,
            )
            rdma.start()
            c = (my - s - 1) % N_DEV
            cp = pltpu.make_async_copy(
                x_hbm.at[pl.ds(c * CHUNK, CHUNK), :], xstage, stage_sem)
            cp.start()
            cp.wait()
            rdma.wait()
            comm[s + 1, :, :] = (
                comm[s + 1, :, :].astype(jnp.float32) + xstage[:, :]
            ).astype(jnp.bfloat16)

        o = (my + 1) % N_DEV
        sv = comm[N_DEV - 1, :, :].astype(jnp.float32)
        r = jnp.maximum(sv, 0.0)
        y = jnp.tanh(sv) * sv * sv + r * r * r
        comm[0, :, :] = y.astype(jnp.bfloat16)
        cp = pltpu.make_async_copy(
            comm.at[0], out_hbm.at[pl.ds(o * CHUNK, CHUNK), :], out_sem)
        cp.start()
        cp.wait()

        pl.semaphore_signal(
            credit, inc=1, device_id=(left,),
            device_id_type=pl.DeviceIdType.MESH,
        )
        pl.semaphore_wait(credit, 1)

        for h in range(1, N_DEV):
            rdma = pltpu.make_async_remote_copy(
                src_ref=comm.at[h - 1],
                dst_ref=comm.at[h],
                send_sem=ag_send.at[h - 1],
                recv_sem=ag_recv.at[h - 1],
                device_id=(right,),
                device_id_type=pl.DeviceIdType.MESH,
            )
            rdma.start()
            rdma.wait()
            c = (my + 1 - h) % N_DEV
            cp = pltpu.make_async_copy(
                comm.at[h], out_hbm.at[pl.ds(c * CHUNK, CHUNK), :], out_sem)
            cp.start()
            cp.wait()

    return pl.pallas_call(
        body,
        out_shape=jax.ShapeDtypeStruct((m_per, n), jnp.bfloat16),
        in_specs=[pl.BlockSpec(memory_space=pltpu.MemorySpace.HBM)],
        out_specs=pl.BlockSpec(memory_space=pltpu.MemorySpace.HBM),
        scratch_shapes=[
            pltpu.VMEM((N_DEV, CHUNK, N_COLS), jnp.bfloat16),
            pltpu.VMEM((CHUNK, N_COLS), jnp.float32),
            pltpu.SemaphoreType.DMA,
            pltpu.SemaphoreType.DMA,
            pltpu.SemaphoreType.DMA((N_DEV - 1,)),
            pltpu.SemaphoreType.DMA((N_DEV - 1,)),
            pltpu.SemaphoreType.DMA((N_DEV - 1,)),
            pltpu.SemaphoreType.DMA((N_DEV - 1,)),
            pltpu.SemaphoreType.REGULAR,
        ],
        compiler_params=pltpu.CompilerParams(
            collective_id=0, vmem_limit_bytes=56 * 1024 * 1024),
    )(t)


# device time: 333695 ns/iter; 1.8068x vs baseline; 1.8068x over previous
import jax
import jax.numpy as jnp
from jax import lax
from jax.experimental import pallas as pl
from jax.experimental.pallas import tpu as pltpu

N_DEV = 4
CHUNK = 2048
N_COLS = 2048
HALF = N_COLS // 2


def _f(s):
    r = jnp.maximum(s, 0.0)
    return jnp.tanh(s) * s * s + r * r * r


def kernel(t):
    m_per, n = t.shape
    assert m_per == N_DEV * CHUNK and n == N_COLS

    def body(x_hbm, out_hbm, comm_cw, comm_ccw, xst_cw, xst_ccw,
             st_sem_cw, st_sem_ccw, out_sem_cw, out_sem_ccw,
             rs_send_cw, rs_recv_cw, ag_send_cw, ag_recv_cw,
             rs_send_ccw, rs_recv_ccw, ag_send_ccw, ag_recv_ccw,
             credit):
        my = lax.axis_index("i")
        left = (my - 1) % N_DEV
        right = (my + 1) % N_DEV

        barrier_sem = pltpu.get_barrier_semaphore()
        for nbr in (left, right):
            pl.semaphore_signal(
                barrier_sem, inc=1,
                device_id=(nbr,), device_id_type=pl.DeviceIdType.MESH,
            )
        pl.semaphore_wait(barrier_sem, 2)

        def stage(c_cw, c_ccw):
            cp1 = pltpu.make_async_copy(
                x_hbm.at[pl.ds(c_cw * CHUNK, CHUNK), pl.ds(0, HALF)],
                xst_cw, st_sem_cw)
            cp2 = pltpu.make_async_copy(
                x_hbm.at[pl.ds(c_ccw * CHUNK, CHUNK), pl.ds(HALF, HALF)],
                xst_ccw, st_sem_ccw)
            cp1.start()
            cp2.start()
            cp1.wait()
            cp2.wait()

        stage(my, my)
        comm_cw[0, :, :] = xst_cw[:, :].astype(jnp.bfloat16)
        comm_ccw[0, :, :] = xst_ccw[:, :].astype(jnp.bfloat16)

        for s in range(N_DEV - 1):
            r_cw = pltpu.make_async_remote_copy(
                src_ref=comm_cw.at[s], dst_ref=comm_cw.at[s + 1],
                send_sem=rs_send_cw.at[s], recv_sem=rs_recv_cw.at[s],
                device_id=(right,), device_id_type=pl.DeviceIdType.MESH,
            )
            r_ccw = pltpu.make_async_remote_copy(
                src_ref=comm_ccw.at[s], dst_ref=comm_ccw.at[s + 1],
                send_sem=rs_send_ccw.at[s], recv_sem=rs_recv_ccw.at[s],
                device_id=(left,), device_id_type=pl.DeviceIdType.MESH,
            )
            r_cw.start()
            r_ccw.start()
            stage((my - s - 1) % N_DEV, (my + s + 1) % N_DEV)
            r_cw.wait()
            comm_cw[s + 1, :, :] = (
                comm_cw[s + 1, :, :].astype(jnp.float32) + xst_cw[:, :]
            ).astype(jnp.bfloat16)
            r_ccw.wait()
            comm_ccw[s + 1, :, :] = (
                comm_ccw[s + 1, :, :].astype(jnp.float32) + xst_ccw[:, :]
            ).astype(jnp.bfloat16)

        o_cw = (my + 1) % N_DEV
        o_ccw = (my - 1) % N_DEV
        comm_cw[0, :, :] = _f(
            comm_cw[N_DEV - 1, :, :].astype(jnp.float32)).astype(jnp.bfloat16)
        comm_ccw[0, :, :] = _f(
            comm_ccw[N_DEV - 1, :, :].astype(jnp.float32)).astype(jnp.bfloat16)
        cp1 = pltpu.make_async_copy(
            comm_cw.at[0],
            out_hbm.at[pl.ds(o_cw * CHUNK, CHUNK), pl.ds(0, HALF)],
            out_sem_cw)
        cp2 = pltpu.make_async_copy(
            comm_ccw.at[0],
            out_hbm.at[pl.ds(o_ccw * CHUNK, CHUNK), pl.ds(HALF, HALF)],
            out_sem_ccw)
        cp1.start()
        cp2.start()
        cp1.wait()
        cp2.wait()

        for nbr in (left, right):
            pl.semaphore_signal(
                credit, inc=1,
                device_id=(nbr,), device_id_type=pl.DeviceIdType.MESH,
            )
        pl.semaphore_wait(credit, 2)

        for h in range(1, N_DEV):
            r_cw = pltpu.make_async_remote_copy(
                src_ref=comm_cw.at[h - 1], dst_ref=comm_cw.at[h],
                send_sem=ag_send_cw.at[h - 1], recv_sem=ag_recv_cw.at[h - 1],
                device_id=(right,), device_id_type=pl.DeviceIdType.MESH,
            )
            r_ccw = pltpu.make_async_remote_copy(
                src_ref=comm_ccw.at[h - 1], dst_ref=comm_ccw.at[h],
                send_sem=ag_send_ccw.at[h - 1], recv_sem=ag_recv_ccw.at[h - 1],
                device_id=(left,), device_id_type=pl.DeviceIdType.MESH,
            )
            r_cw.start()
            r_ccw.start()
            r_cw.wait()
            r_ccw.wait()
            c_cw = (my + 1 - h) % N_DEV
            c_ccw = (my - 1 + h) % N_DEV
            cp1 = pltpu.make_async_copy(
                comm_cw.at[h],
                out_hbm.at[pl.ds(c_cw * CHUNK, CHUNK), pl.ds(0, HALF)],
                out_sem_cw)
            cp2 = pltpu.make_async_copy(
                comm_ccw.at[h],
                out_hbm.at[pl.ds(c_ccw * CHUNK, CHUNK), pl.ds(HALF, HALF)],
                out_sem_ccw)
            cp1.start()
            cp2.start()
            cp1.wait()
            cp2.wait()

    return pl.pallas_call(
        body,
        out_shape=jax.ShapeDtypeStruct((m_per, n), jnp.bfloat16),
        in_specs=[pl.BlockSpec(memory_space=pltpu.MemorySpace.HBM)],
        out_specs=pl.BlockSpec(memory_space=pltpu.MemorySpace.HBM),
        scratch_shapes=[
            pltpu.VMEM((N_DEV, CHUNK, HALF), jnp.bfloat16),
            pltpu.VMEM((N_DEV, CHUNK, HALF), jnp.bfloat16),
            pltpu.VMEM((CHUNK, HALF), jnp.float32),
            pltpu.VMEM((CHUNK, HALF), jnp.float32),
            pltpu.SemaphoreType.DMA,
            pltpu.SemaphoreType.DMA,
            pltpu.SemaphoreType.DMA,
            pltpu.SemaphoreType.DMA,
            pltpu.SemaphoreType.DMA((N_DEV - 1,)),
            pltpu.SemaphoreType.DMA((N_DEV - 1,)),
            pltpu.SemaphoreType.DMA((N_DEV - 1,)),
            pltpu.SemaphoreType.DMA((N_DEV - 1,)),
            pltpu.SemaphoreType.DMA((N_DEV - 1,)),
            pltpu.SemaphoreType.DMA((N_DEV - 1,)),
            pltpu.SemaphoreType.DMA((N_DEV - 1,)),
            pltpu.SemaphoreType.DMA((N_DEV - 1,)),
            pltpu.SemaphoreType.REGULAR,
        ],
        compiler_params=pltpu.CompilerParams(
            collective_id=0, vmem_limit_bytes=56 * 1024 * 1024),
    )(t)


# device time: 309104 ns/iter; 1.9506x vs baseline; 1.0796x over previous
import jax
import jax.numpy as jnp
from jax import lax
from jax.experimental import pallas as pl
from jax.experimental.pallas import tpu as pltpu

N_DEV = 4
CHUNK = 2048
SUB = CHUNK // 2
N_COLS = 2048
HALF = N_COLS // 2


def _f(s):
    r = jnp.maximum(s, 0.0)
    return jnp.tanh(s) * s * s + r * r * r


def kernel(t):
    m_per, n = t.shape
    assert m_per == N_DEV * CHUNK and n == N_COLS

    def body(x_hbm, out_hbm, comm_cw, comm_ccw, xst_cw, xst_ccw,
             st_sem_cw, st_sem_ccw, out_sem_cw, out_sem_ccw,
             rs_send_cw, rs_recv_cw, ag_send_cw, ag_recv_cw,
             rs_send_ccw, rs_recv_ccw, ag_send_ccw, ag_recv_ccw):
        my = lax.axis_index("i")
        left = (my - 1) % N_DEV
        right = (my + 1) % N_DEV
        pending = []

        barrier_sem = pltpu.get_barrier_semaphore()
        for nbr in (left, right):
            pl.semaphore_signal(
                barrier_sem, inc=1,
                device_id=(nbr,), device_id_type=pl.DeviceIdType.MESH,
            )
        pl.semaphore_wait(barrier_sem, 2)

        def stage_start(c_cw, c_ccw):
            cp1 = pltpu.make_async_copy(
                x_hbm.at[pl.ds(c_cw * CHUNK, CHUNK), pl.ds(0, HALF)],
                xst_cw, st_sem_cw)
            cp2 = pltpu.make_async_copy(
                x_hbm.at[pl.ds(c_ccw * CHUNK, CHUNK), pl.ds(HALF, HALF)],
                xst_ccw, st_sem_ccw)
            cp1.start()
            cp2.start()
            return cp1, cp2

        def rs_rdma(comm, sends, recvs, s, u, dev):
            return pltpu.make_async_remote_copy(
                src_ref=comm.at[s, pl.ds(u * SUB, SUB), :],
                dst_ref=comm.at[s + 1, pl.ds(u * SUB, SUB), :],
                send_sem=sends.at[s, u], recv_sem=recvs.at[s, u],
                device_id=(dev,), device_id_type=pl.DeviceIdType.MESH,
            )

        def ag_rdma(sends, recvs, h, u, c, col0, dev):
            rows = out_hbm.at[pl.ds(c * CHUNK + u * SUB, SUB),
                              pl.ds(col0, HALF)]
            return pltpu.make_async_remote_copy(
                src_ref=rows, dst_ref=rows,
                send_sem=sends.at[h - 1, u], recv_sem=recvs.at[h - 1, u],
                device_id=(dev,), device_id_type=pl.DeviceIdType.MESH,
            )

        cp1, cp2 = stage_start(my, my)
        cp1.wait()
        cp2.wait()
        comm_cw[0, :, :] = xst_cw[:, :].astype(jnp.bfloat16)
        comm_ccw[0, :, :] = xst_ccw[:, :].astype(jnp.bfloat16)

        for u in (0, 1):
            r = rs_rdma(comm_cw, rs_send_cw, rs_recv_cw, 0, u, right)
            r.start()
            pending.append(r)
            r = rs_rdma(comm_ccw, rs_send_ccw, rs_recv_ccw, 0, u, left)
            r.start()
            pending.append(r)
        cp1, cp2 = stage_start((my - 1) % N_DEV, (my + 1) % N_DEV)

        for s in range(N_DEV - 1):
            cp1.wait()
            cp2.wait()
            for u in (0, 1):
                rows = pl.ds(u * SUB, SUB)
                rs_rdma(comm_cw, rs_send_cw, rs_recv_cw, s, u,
                        right).wait_recv()
                comm_cw[s + 1, rows, :] = (
                    comm_cw[s + 1, rows, :].astype(jnp.float32)
                    + xst_cw[rows, :]
                ).astype(jnp.bfloat16)
                if s < N_DEV - 2:
                    r = rs_rdma(comm_cw, rs_send_cw, rs_recv_cw, s + 1, u,
                                right)
                    r.start()
                    pending.append(r)
                rs_rdma(comm_ccw, rs_send_ccw, rs_recv_ccw, s, u,
                        left).wait_recv()
                comm_ccw[s + 1, rows, :] = (
                    comm_ccw[s + 1, rows, :].astype(jnp.float32)
                    + xst_ccw[rows, :]
                ).astype(jnp.bfloat16)
                if s < N_DEV - 2:
                    r = rs_rdma(comm_ccw, rs_send_ccw, rs_recv_ccw, s + 1, u,
                                left)
                    r.start()
                    pending.append(r)
            if s < N_DEV - 2:
                cp1, cp2 = stage_start((my - s - 2) % N_DEV,
                                       (my + s + 2) % N_DEV)

        o_cw = (my + 1) % N_DEV
        o_ccw = (my - 1) % N_DEV
        for u in (0, 1):
            rows = pl.ds(u * SUB, SUB)
            comm_cw[N_DEV - 1, rows, :] = _f(
                comm_cw[N_DEV - 1, rows, :].astype(jnp.float32)
            ).astype(jnp.bfloat16)
            comm_ccw[N_DEV - 1, rows, :] = _f(
                comm_ccw[N_DEV - 1, rows, :].astype(jnp.float32)
            ).astype(jnp.bfloat16)
            cp1 = pltpu.make_async_copy(
                comm_cw.at[N_DEV - 1, rows, :],
                out_hbm.at[pl.ds(o_cw * CHUNK + u * SUB, SUB),
                           pl.ds(0, HALF)],
                out_sem_cw)
            cp2 = pltpu.make_async_copy(
                comm_ccw.at[N_DEV - 1, rows, :],
                out_hbm.at[pl.ds(o_ccw * CHUNK + u * SUB, SUB),
                           pl.ds(HALF, HALF)],
                out_sem_ccw)
            cp1.start()
            cp2.start()
            cp1.wait()
            r = ag_rdma(ag_send_cw, ag_recv_cw, 1, u, o_cw, 0, right)
            r.start()
            pending.append(r)
            cp2.wait()
            r = ag_rdma(ag_send_ccw, ag_recv_ccw, 1, u, o_ccw, HALF, left)
            r.start()
            pending.append(r)

        for h in range(1, N_DEV):
            c_cw = (my + 1 - h) % N_DEV
            c_ccw = (my - 1 + h) % N_DEV
            for u in (0, 1):
                ag_rdma(ag_send_cw, ag_recv_cw, h, u, c_cw, 0,
                        right).wait_recv()
                if h < N_DEV - 1:
                    r = ag_rdma(ag_send_cw, ag_recv_cw, h + 1, u, c_cw, 0,
                                right)
                    r.start()
                    pending.append(r)
                ag_rdma(ag_send_ccw, ag_recv_ccw, h, u, c_ccw, HALF,
                        left).wait_recv()
                if h < N_DEV - 1:
                    r = ag_rdma(ag_send_ccw, ag_recv_ccw, h + 1, u, c_ccw,
                                HALF, left)
                    r.start()
                    pending.append(r)

        for r in pending:
            r.wait_send()

    return pl.pallas_call(
        body,
        out_shape=jax.ShapeDtypeStruct((m_per, n), jnp.bfloat16),
        in_specs=[pl.BlockSpec(memory_space=pltpu.MemorySpace.HBM)],
        out_specs=pl.BlockSpec(memory_space=pltpu.MemorySpace.HBM),
        scratch_shapes=[
            pltpu.VMEM((N_DEV, CHUNK, HALF), jnp.bfloat16),
            pltpu.VMEM((N_DEV, CHUNK, HALF), jnp.bfloat16),
            pltpu.VMEM((CHUNK, HALF), jnp.float32),
            pltpu.VMEM((CHUNK, HALF), jnp.float32),
            pltpu.SemaphoreType.DMA,
            pltpu.SemaphoreType.DMA,
            pltpu.SemaphoreType.DMA,
            pltpu.SemaphoreType.DMA,
            pltpu.SemaphoreType.DMA((N_DEV - 1, 2)),
            pltpu.SemaphoreType.DMA((N_DEV - 1, 2)),
            pltpu.SemaphoreType.DMA((N_DEV - 1, 2)),
            pltpu.SemaphoreType.DMA((N_DEV - 1, 2)),
            pltpu.SemaphoreType.DMA((N_DEV - 1, 2)),
            pltpu.SemaphoreType.DMA((N_DEV - 1, 2)),
            pltpu.SemaphoreType.DMA((N_DEV - 1, 2)),
            pltpu.SemaphoreType.DMA((N_DEV - 1, 2)),
        ],
        compiler_params=pltpu.CompilerParams(
            collective_id=0, vmem_limit_bytes=56 * 1024 * 1024),
    )(t)


# device time: 307235 ns/iter; 1.9625x vs baseline; 1.0061x over previous
import jax
import jax.numpy as jnp
from jax import lax
from jax.experimental import pallas as pl
from jax.experimental.pallas import tpu as pltpu

N_DEV = 4
CHUNK = 2048
N_SUB = 4
SUB = CHUNK // N_SUB
N_COLS = 2048
HALF = N_COLS // 2


def _f(s):
    r = jnp.maximum(s, 0.0)
    return jnp.tanh(s) * s * s + r * r * r


def kernel(t):
    m_per, n = t.shape
    assert m_per == N_DEV * CHUNK and n == N_COLS

    def body(x_hbm, out_hbm, comm_cw, comm_ccw, xst_cw, xst_ccw,
             st_sem_cw, st_sem_ccw, out_sem_cw, out_sem_ccw,
             rs_send_cw, rs_recv_cw, ag_send_cw, ag_recv_cw,
             rs_send_ccw, rs_recv_ccw, ag_send_ccw, ag_recv_ccw):
        my = lax.axis_index("i")
        left = (my - 1) % N_DEV
        right = (my + 1) % N_DEV
        pending = []

        barrier_sem = pltpu.get_barrier_semaphore()
        for nbr in (left, right):
            pl.semaphore_signal(
                barrier_sem, inc=1,
                device_id=(nbr,), device_id_type=pl.DeviceIdType.MESH,
            )
        pl.semaphore_wait(barrier_sem, 2)

        def stage_start(c_cw, c_ccw):
            cp1 = pltpu.make_async_copy(
                x_hbm.at[pl.ds(c_cw * CHUNK, CHUNK), pl.ds(0, HALF)],
                xst_cw, st_sem_cw)
            cp2 = pltpu.make_async_copy(
                x_hbm.at[pl.ds(c_ccw * CHUNK, CHUNK), pl.ds(HALF, HALF)],
                xst_ccw, st_sem_ccw)
            cp1.start()
            cp2.start()
            return cp1, cp2

        def rs_rdma(comm, sends, recvs, s, u, dev):
            return pltpu.make_async_remote_copy(
                src_ref=comm.at[s, pl.ds(u * SUB, SUB), :],
                dst_ref=comm.at[s + 1, pl.ds(u * SUB, SUB), :],
                send_sem=sends.at[s, u], recv_sem=recvs.at[s, u],
                device_id=(dev,), device_id_type=pl.DeviceIdType.MESH,
            )

        def ag_rdma(sends, recvs, h, u, c, col0, dev):
            rows = out_hbm.at[pl.ds(c * CHUNK + u * SUB, SUB),
                              pl.ds(col0, HALF)]
            return pltpu.make_async_remote_copy(
                src_ref=rows, dst_ref=rows,
                send_sem=sends.at[h - 1, u], recv_sem=recvs.at[h - 1, u],
                device_id=(dev,), device_id_type=pl.DeviceIdType.MESH,
            )

        cp1, cp2 = stage_start(my, my)
        cp1.wait()
        cp2.wait()
        for u in range(N_SUB):
            rows = pl.ds(u * SUB, SUB)
            comm_cw[0, rows, :] = xst_cw[rows, :].astype(jnp.bfloat16)
            r = rs_rdma(comm_cw, rs_send_cw, rs_recv_cw, 0, u, right)
            r.start()
            pending.append(r)
            comm_ccw[0, rows, :] = xst_ccw[rows, :].astype(jnp.bfloat16)
            r = rs_rdma(comm_ccw, rs_send_ccw, rs_recv_ccw, 0, u, left)
            r.start()
            pending.append(r)
        cp1, cp2 = stage_start((my - 1) % N_DEV, (my + 1) % N_DEV)

        for s in range(N_DEV - 1):
            cp1.wait()
            cp2.wait()
            for u in range(N_SUB):
                rows = pl.ds(u * SUB, SUB)
                rs_rdma(comm_cw, rs_send_cw, rs_recv_cw, s, u,
                        right).wait_recv()
                comm_cw[s + 1, rows, :] = (
                    comm_cw[s + 1, rows, :].astype(jnp.float32)
                    + xst_cw[rows, :]
                ).astype(jnp.bfloat16)
                if s < N_DEV - 2:
                    r = rs_rdma(comm_cw, rs_send_cw, rs_recv_cw, s + 1, u,
                                right)
                    r.start()
                    pending.append(r)
                rs_rdma(comm_ccw, rs_send_ccw, rs_recv_ccw, s, u,
                        left).wait_recv()
                comm_ccw[s + 1, rows, :] = (
                    comm_ccw[s + 1, rows, :].astype(jnp.float32)
                    + xst_ccw[rows, :]
                ).astype(jnp.bfloat16)
                if s < N_DEV - 2:
                    r = rs_rdma(comm_ccw, rs_send_ccw, rs_recv_ccw, s + 1, u,
                                left)
                    r.start()
                    pending.append(r)
            if s < N_DEV - 2:
                cp1, cp2 = stage_start((my - s - 2) % N_DEV,
                                       (my + s + 2) % N_DEV)

        o_cw = (my + 1) % N_DEV
        o_ccw = (my - 1) % N_DEV
        for u in range(N_SUB):
            rows = pl.ds(u * SUB, SUB)
            comm_cw[N_DEV - 1, rows, :] = _f(
                comm_cw[N_DEV - 1, rows, :].astype(jnp.float32)
            ).astype(jnp.bfloat16)
            comm_ccw[N_DEV - 1, rows, :] = _f(
                comm_ccw[N_DEV - 1, rows, :].astype(jnp.float32)
            ).astype(jnp.bfloat16)
            cp1 = pltpu.make_async_copy(
                comm_cw.at[N_DEV - 1, rows, :],
                out_hbm.at[pl.ds(o_cw * CHUNK + u * SUB, SUB),
                           pl.ds(0, HALF)],
                out_sem_cw)
            cp2 = pltpu.make_async_copy(
                comm_ccw.at[N_DEV - 1, rows, :],
                out_hbm.at[pl.ds(o_ccw * CHUNK + u * SUB, SUB),
                           pl.ds(HALF, HALF)],
                out_sem_ccw)
            cp1.start()
            cp2.start()
            cp1.wait()
            r = ag_rdma(ag_send_cw, ag_recv_cw, 1, u, o_cw, 0, right)
            r.start()
            pending.append(r)
            cp2.wait()
            r = ag_rdma(ag_send_ccw, ag_recv_ccw, 1, u, o_ccw, HALF, left)
            r.start()
            pending.append(r)

        for h in range(1, N_DEV):
            c_cw = (my + 1 - h) % N_DEV
            c_ccw = (my - 1 + h) % N_DEV
            for u in range(N_SUB):
                ag_rdma(ag_send_cw, ag_recv_cw, h, u, c_cw, 0,
                        right).wait_recv()
                if h < N_DEV - 1:
                    r = ag_rdma(ag_send_cw, ag_recv_cw, h + 1, u, c_cw, 0,
                                right)
                    r.start()
                    pending.append(r)
                ag_rdma(ag_send_ccw, ag_recv_ccw, h, u, c_ccw, HALF,
                        left).wait_recv()
                if h < N_DEV - 1:
                    r = ag_rdma(ag_send_ccw, ag_recv_ccw, h + 1, u, c_ccw,
                                HALF, left)
                    r.start()
                    pending.append(r)

        for r in pending:
            r.wait_send()

    return pl.pallas_call(
        body,
        out_shape=jax.ShapeDtypeStruct((m_per, n), jnp.bfloat16),
        in_specs=[pl.BlockSpec(memory_space=pltpu.MemorySpace.HBM)],
        out_specs=pl.BlockSpec(memory_space=pltpu.MemorySpace.HBM),
        scratch_shapes=[
            pltpu.VMEM((N_DEV, CHUNK, HALF), jnp.bfloat16),
            pltpu.VMEM((N_DEV, CHUNK, HALF), jnp.bfloat16),
            pltpu.VMEM((CHUNK, HALF), jnp.float32),
            pltpu.VMEM((CHUNK, HALF), jnp.float32),
            pltpu.SemaphoreType.DMA,
            pltpu.SemaphoreType.DMA,
            pltpu.SemaphoreType.DMA,
            pltpu.SemaphoreType.DMA,
            pltpu.SemaphoreType.DMA((N_DEV - 1, N_SUB)),
            pltpu.SemaphoreType.DMA((N_DEV - 1, N_SUB)),
            pltpu.SemaphoreType.DMA((N_DEV - 1, N_SUB)),
            pltpu.SemaphoreType.DMA((N_DEV - 1, N_SUB)),
            pltpu.SemaphoreType.DMA((N_DEV - 1, N_SUB)),
            pltpu.SemaphoreType.DMA((N_DEV - 1, N_SUB)),
            pltpu.SemaphoreType.DMA((N_DEV - 1, N_SUB)),
            pltpu.SemaphoreType.DMA((N_DEV - 1, N_SUB)),
        ],
        compiler_params=pltpu.CompilerParams(
            collective_id=0, vmem_limit_bytes=56 * 1024 * 1024),
    )(t)
